# Initial kernel scaffold; baseline (speedup 1.0000x reference)
import jax
import jax.numpy as jnp
from jax.experimental import pallas as pl


def kernel(x, edge_index, batch, W0, b0, bn0_w, bn0_b, res0_W, res0_b, W1, b1, bn1_w, bn1_b, Wh, bh):
    raise NotImplementedError("write your pallas kernel here")


# speedup vs baseline: 9.1371x; 9.1371x over previous
import functools

import jax
import jax.numpy as jnp
from jax import lax
from jax.experimental import pallas as pl
from jax.experimental.pallas import tpu as pltpu
from jax.experimental.pallas import tpu_sc as plsc

N = 10000
E = 320000
IN_DIM = 128
HIDDEN = 256
QW = 64
NQ = 4
NUM_CLASSES = 64
NUM_GRAPHS = 64
EPS = 1e-5

NC = 2
NS = 16
CH = 80
NCHUNK = E // CH
ROWS_PER_SUB = 624
ROWS_TAIL = N - NS * ROWS_PER_SUB

_HIGH = jax.lax.Precision.HIGHEST


def _ranged_copy(copy_fn, sid):
    copy_fn(sid * ROWS_PER_SUB, ROWS_PER_SUB)

    @pl.when(sid == NS - 1)
    def _():
        copy_fn(NS * ROWS_PER_SUB, ROWS_TAIL)


def _mesh():
    return plsc.VectorSubcoreMesh(core_axis_name="c", subcore_axis_name="s")


def _sc_degree(dst_deg, zeros8, ones_ch):
    cpw = NCHUNK // NC // NS

    @functools.partial(
        pl.kernel,
        mesh=_mesh(),
        compiler_params=pltpu.CompilerParams(use_tc_tiling_on_sc=False),
        out_type=jax.ShapeDtypeStruct((NC, N, 8), jnp.float32),
        scratch_types=[
            pltpu.VMEM((cpw, CH), jnp.int32),
            pltpu.VMEM((CH, 8), jnp.float32),
            pltpu.VMEM_SHARED((N, 8), jnp.float32),
        ],
    )
    def k(dst_hbm, z_hbm, ones_hbm, out_hbm, idx_v, ones_v, acc_sh):
        cid = lax.axis_index("c")
        sid = lax.axis_index("s")
        _ranged_copy(lambda o, n: pltpu.sync_copy(
            z_hbm.at[pl.ds(o, n)], acc_sh.at[pl.ds(o, n)]), sid)
        pltpu.sync_copy(dst_hbm.at[cid * NS + sid], idx_v)
        pltpu.sync_copy(ones_hbm, ones_v)
        plsc.subcore_barrier()

        @pl.loop(0, cpw)
        def _(c):
            pltpu.sync_copy(ones_v, acc_sh.at[idx_v.at[c]], add=True)

        plsc.subcore_barrier()
        _ranged_copy(lambda o, n: pltpu.sync_copy(
            acc_sh.at[pl.ds(o, n)], out_hbm.at[cid, pl.ds(o, n)]), sid)

    return k(dst_deg, zeros8, ones_ch)


def _sc_conv(g4n, src_st, dst_conv):
    cpw = NCHUNK // NS

    @functools.partial(
        pl.kernel,
        mesh=_mesh(),
        compiler_params=pltpu.CompilerParams(use_tc_tiling_on_sc=False),
        out_type=jax.ShapeDtypeStruct((NQ, N, QW), jnp.float32),
        scratch_types=[
            pltpu.VMEM((cpw, CH), jnp.int32),
            pltpu.VMEM((cpw, CH), jnp.int32),
            pltpu.VMEM((CH, QW), jnp.float32),
            pltpu.VMEM_SHARED((N, QW), jnp.float32),
        ],
    )
    def k(g_hbm, src_hbm, dst_hbm, out_hbm, src_v, dst_v, rows_v, acc_sh):
        cid = lax.axis_index("c")
        sid = lax.axis_index("s")
        pltpu.sync_copy(dst_hbm.at[sid], dst_v)
        for p in range(NQ // NC):
            q = cid * (NQ // NC) + p
            _ranged_copy(lambda o, n: pltpu.sync_copy(
                g_hbm.at[pl.ds(q * N + o, n)], acc_sh.at[pl.ds(o, n)]), sid)
            pltpu.sync_copy(src_hbm.at[q, sid], src_v)
            plsc.subcore_barrier()

            @pl.loop(0, cpw)
            def _(c):
                pltpu.sync_copy(g_hbm.at[src_v.at[c]], rows_v)
                pltpu.sync_copy(rows_v, acc_sh.at[dst_v.at[c]], add=True)

            plsc.subcore_barrier()
            _ranged_copy(lambda o, n: pltpu.sync_copy(
                acc_sh.at[pl.ds(o, n)], out_hbm.at[q, pl.ds(o, n)]), sid)

    return k(g4n, src_st, dst_conv)


GB = 2000
NBLK = N // GB


def _tc_dinv(degp):
    def body(degp_ref, dinv_ref):
        deg = degp_ref[0, :, 0:1] + degp_ref[1, :, 0:1] + 1.0
        dinv_ref[...] = jnp.broadcast_to(jax.lax.rsqrt(deg), (N, QW))

    return pl.pallas_call(
        body, out_shape=jax.ShapeDtypeStruct((N, QW), jnp.float32))(degp)


def _split_quarters(g_ref, dv, h):
    for q in range(NQ):
        g_ref[q] = dv * h[:, q * QW:(q + 1) * QW]


def _tc_prep(x, W0, res0_W, res0_b, dinv):
    def body(x_ref, w_ref, rw_ref, rb_ref, dinv_ref, g_ref, res_ref):
        xv = x_ref[...]
        h = lax.dot_general(xv, w_ref[...], (((1,), (0,)), ((), ())),
                            precision=_HIGH, preferred_element_type=jnp.float32)
        _split_quarters(g_ref, dinv_ref[...], h)
        res_ref[...] = lax.dot_general(
            xv, rw_ref[...], (((1,), (1,)), ((), ())),
            precision=_HIGH, preferred_element_type=jnp.float32) + rb_ref[...]

    return pl.pallas_call(
        body,
        grid=(NBLK,),
        in_specs=[
            pl.BlockSpec((GB, IN_DIM), lambda i: (i, 0)),
            pl.BlockSpec((IN_DIM, HIDDEN), lambda i: (0, 0)),
            pl.BlockSpec((HIDDEN, IN_DIM), lambda i: (0, 0)),
            pl.BlockSpec((HIDDEN,), lambda i: (0,)),
            pl.BlockSpec((GB, QW), lambda i: (i, 0)),
        ],
        out_specs=[
            pl.BlockSpec((NQ, GB, QW), lambda i: (0, i, 0)),
            pl.BlockSpec((GB, HIDDEN), lambda i: (i, 0)),
        ],
        out_shape=[
            jax.ShapeDtypeStruct((NQ, N, QW), jnp.float32),
            jax.ShapeDtypeStruct((N, HIDDEN), jnp.float32),
        ])(x, W0, res0_W, res0_b, dinv)


def _tc_bn_stats(s, dinv, b):
    def body(s_ref, dinv_ref, b_ref, out0_ref, st_ref):
        dv = dinv_ref[...]
        out = jnp.concatenate([dv * s_ref[q] for q in range(NQ)],
                              axis=1) + b_ref[...]
        out0_ref[...] = out

        @pl.when(pl.program_id(0) == 0)
        def _():
            st_ref[...] = jnp.zeros_like(st_ref)

        st_ref[0:1, :] += jnp.sum(out, axis=0, keepdims=True)
        st_ref[1:2, :] += jnp.sum(out * out, axis=0, keepdims=True)

    return pl.pallas_call(
        body,
        grid=(NBLK,),
        in_specs=[
            pl.BlockSpec((NQ, GB, QW), lambda i: (0, i, 0)),
            pl.BlockSpec((GB, QW), lambda i: (i, 0)),
            pl.BlockSpec((HIDDEN,), lambda i: (0,)),
        ],
        out_specs=[
            pl.BlockSpec((GB, HIDDEN), lambda i: (i, 0)),
            pl.BlockSpec((2, HIDDEN), lambda i: (0, 0)),
        ],
        out_shape=[
            jax.ShapeDtypeStruct((N, HIDDEN), jnp.float32),
            jax.ShapeDtypeStruct((2, HIDDEN), jnp.float32),
        ])(s, dinv, b)


def _tc_bn_apply(out0, stats, bn_w, bn_b, res):
    def body(o_ref, st_ref, w_ref, bb_ref, res_ref, x1_ref):
        m = st_ref[0:1, :] * (1.0 / N)
        var = st_ref[1:2, :] * (1.0 / N) - m * m
        xb = (o_ref[...] - m) * jax.lax.rsqrt(var + EPS) * w_ref[...] + bb_ref[...]
        x1_ref[...] = jnp.maximum(xb, 0.0) + res_ref[...]

    return pl.pallas_call(
        body,
        grid=(NBLK,),
        in_specs=[
            pl.BlockSpec((GB, HIDDEN), lambda i: (i, 0)),
            pl.BlockSpec((2, HIDDEN), lambda i: (0, 0)),
            pl.BlockSpec((HIDDEN,), lambda i: (0,)),
            pl.BlockSpec((HIDDEN,), lambda i: (0,)),
            pl.BlockSpec((GB, HIDDEN), lambda i: (i, 0)),
        ],
        out_specs=pl.BlockSpec((GB, HIDDEN), lambda i: (i, 0)),
        out_shape=jax.ShapeDtypeStruct((N, HIDDEN), jnp.float32))(
            out0, stats, bn_w, bn_b, res)


def _tc_mm(x1, W1, dinv):
    def body(x_ref, w_ref, dinv_ref, g_ref):
        h = lax.dot_general(x_ref[...], w_ref[...], (((1,), (0,)), ((), ())),
                            precision=_HIGH, preferred_element_type=jnp.float32)
        _split_quarters(g_ref, dinv_ref[...], h)

    return pl.pallas_call(
        body,
        grid=(NBLK,),
        in_specs=[
            pl.BlockSpec((GB, HIDDEN), lambda i: (i, 0)),
            pl.BlockSpec((HIDDEN, HIDDEN), lambda i: (0, 0)),
            pl.BlockSpec((GB, QW), lambda i: (i, 0)),
        ],
        out_specs=pl.BlockSpec((NQ, GB, QW), lambda i: (0, i, 0)),
        out_shape=jax.ShapeDtypeStruct((NQ, N, QW), jnp.float32))(x1, W1, dinv)


def _tc_head(x2, Wh, bh, batch2d):
    def body(x2_ref, wh_ref, bh_ref, b2_ref, out_ref):
        logits = lax.dot_general(
            x2_ref[...], wh_ref[...], (((1,), (1,)), ((), ())),
            precision=_HIGH, preferred_element_type=jnp.float32) + bh_ref[...]
        seg = lax.broadcasted_iota(jnp.int32, (NUM_GRAPHS, N), 0)
        P = (b2_ref[...] == seg).astype(jnp.float32)
        pooled = lax.dot_general(P, logits, (((1,), (0,)), ((), ())),
                                 precision=_HIGH,
                                 preferred_element_type=jnp.float32)
        cnt = jnp.clip(jnp.sum(P, axis=1, keepdims=True), 1.0, None)
        out_ref[...] = pooled / cnt

    return pl.pallas_call(
        body,
        out_shape=jax.ShapeDtypeStruct((NUM_GRAPHS, NUM_CLASSES), jnp.float32))(
            x2, Wh, bh, batch2d)


def kernel(x, edge_index, batch, W0, b0, bn0_w, bn0_b, res0_W, res0_b,
           W1, b1, bn1_w, bn1_b, Wh, bh):
    src = edge_index[0].astype(jnp.int32)
    dst = edge_index[1].astype(jnp.int32)
    src_st = jnp.stack([src + q * N for q in range(NQ)]).reshape(
        NQ, NS, NCHUNK // NS, CH)
    dst_conv = dst.reshape(NS, NCHUNK // NS, CH)
    dst_deg = dst.reshape(NC * NS, NCHUNK // (NC * NS), CH)
    zeros8 = jnp.zeros((N, 8), jnp.float32)
    ones_ch = jnp.ones((CH, 8), jnp.float32)

    degp = _sc_degree(dst_deg, zeros8, ones_ch)
    dinv = _tc_dinv(degp)
    g0, res = _tc_prep(x, W0, res0_W, res0_b, dinv)
    s0 = _sc_conv(g0.reshape(NQ * N, QW), src_st, dst_conv)
    out0, st0 = _tc_bn_stats(s0, dinv, b0)
    x1 = _tc_bn_apply(out0, st0, bn0_w, bn0_b, res)
    g1 = _tc_mm(x1, W1, dinv)
    s1 = _sc_conv(g1.reshape(NQ * N, QW), src_st, dst_conv)
    out1, st1 = _tc_bn_stats(s1, dinv, b1)
    x2 = _tc_bn_apply(out1, st1, bn1_w, bn1_b, x1)
    return _tc_head(x2, Wh, bh, batch.astype(jnp.int32).reshape(1, N))


# speedup vs baseline: 14.6070x; 1.5987x over previous
import functools

import jax
import jax.numpy as jnp
from jax import lax
from jax.experimental import pallas as pl
from jax.experimental.pallas import tpu as pltpu
from jax.experimental.pallas import tpu_sc as plsc

N = 10000
E = 320000
IN_DIM = 128
HIDDEN = 256
QW = 64
NQ = 4
NUM_CLASSES = 64
NUM_GRAPHS = 64
EPS = 1e-5

NC = 2
NS = 16
CH = 80
NCHUNK = E // CH
ROWS_PER_SUB = 624
ROWS_TAIL = N - NS * ROWS_PER_SUB

_HIGH = jax.lax.Precision.HIGHEST


def _ranged_copy(copy_fn, sid):
    copy_fn(sid * ROWS_PER_SUB, ROWS_PER_SUB)

    @pl.when(sid == NS - 1)
    def _():
        copy_fn(NS * ROWS_PER_SUB, ROWS_TAIL)


def _mesh():
    return plsc.VectorSubcoreMesh(core_axis_name="c", subcore_axis_name="s")


def _sc_degree(dst_deg, zeros8, ones_ch):
    cpw = NCHUNK // NC // NS

    @functools.partial(
        pl.kernel,
        mesh=_mesh(),
        compiler_params=pltpu.CompilerParams(use_tc_tiling_on_sc=False),
        out_type=jax.ShapeDtypeStruct((NC, N, 8), jnp.float32),
        scratch_types=[
            pltpu.VMEM((cpw, CH), jnp.int32),
            pltpu.VMEM((CH, 8), jnp.float32),
            pltpu.VMEM_SHARED((N, 8), jnp.float32),
        ],
    )
    def k(dst_hbm, z_hbm, ones_hbm, out_hbm, idx_v, ones_v, acc_sh):
        cid = lax.axis_index("c")
        sid = lax.axis_index("s")
        _ranged_copy(lambda o, n: pltpu.sync_copy(
            z_hbm.at[pl.ds(o, n)], acc_sh.at[pl.ds(o, n)]), sid)
        pltpu.sync_copy(dst_hbm.at[cid * NS + sid], idx_v)
        pltpu.sync_copy(ones_hbm, ones_v)
        plsc.subcore_barrier()

        @pl.loop(0, cpw)
        def _(c):
            pltpu.sync_copy(ones_v, acc_sh.at[idx_v.at[c]], add=True)

        plsc.subcore_barrier()
        _ranged_copy(lambda o, n: pltpu.sync_copy(
            acc_sh.at[pl.ds(o, n)], out_hbm.at[cid, pl.ds(o, n)]), sid)

    return k(dst_deg, zeros8, ones_ch)


def _sc_conv(g4n, src_st, dst_conv):
    cpw = NCHUNK // NS

    @functools.partial(
        pl.kernel,
        mesh=_mesh(),
        compiler_params=pltpu.CompilerParams(use_tc_tiling_on_sc=False),
        out_type=jax.ShapeDtypeStruct((NQ, N, QW), jnp.float32),
        scratch_types=[
            pltpu.VMEM((cpw, CH), jnp.int32),
            pltpu.VMEM((cpw, CH), jnp.int32),
            pltpu.VMEM((CH, QW), jnp.float32),
            pltpu.VMEM((CH, QW), jnp.float32),
            pltpu.VMEM_SHARED((N, QW), jnp.float32),
            pltpu.SemaphoreType.DMA,
            pltpu.SemaphoreType.DMA,
            pltpu.SemaphoreType.DMA,
            pltpu.SemaphoreType.DMA,
        ],
    )
    def k(g_hbm, src_hbm, dst_hbm, out_hbm, src_v, dst_v, rows0, rows1,
          acc_sh, gs0, gs1, ss0, ss1):
        cid = lax.axis_index("c")
        sid = lax.axis_index("s")
        pltpu.sync_copy(dst_hbm.at[sid], dst_v)
        for p in range(NQ // NC):
            q = cid * (NQ // NC) + p
            _ranged_copy(lambda o, n: pltpu.sync_copy(
                g_hbm.at[pl.ds(q * N + o, n)], acc_sh.at[pl.ds(o, n)]), sid)
            pltpu.sync_copy(src_hbm.at[q, sid], src_v)
            plsc.subcore_barrier()

            pltpu.async_copy(g_hbm.at[src_v.at[0]], rows0, gs0)
            pltpu.async_copy(g_hbm.at[src_v.at[1]], rows1, gs1)

            @pl.loop(0, cpw // 2)
            def _(i):
                c = i * 2
                for b, (buf, gs, ss) in enumerate(
                        ((rows0, gs0, ss0), (rows1, gs1, ss1))):
                    cc = c + b
                    pltpu.make_async_copy(g_hbm.at[src_v.at[0]], buf, gs).wait()
                    pltpu.async_copy(buf, acc_sh.at[dst_v.at[cc]], ss, add=True)

                    @pl.when(cc + 2 < cpw)
                    def _():
                        pltpu.make_async_copy(
                            buf, acc_sh.at[dst_v.at[0]], ss).wait()
                        pltpu.async_copy(g_hbm.at[src_v.at[cc + 2]], buf, gs)

            pltpu.make_async_copy(rows0, acc_sh.at[dst_v.at[0]], ss0).wait()
            pltpu.make_async_copy(rows1, acc_sh.at[dst_v.at[0]], ss1).wait()
            plsc.subcore_barrier()
            _ranged_copy(lambda o, n: pltpu.sync_copy(
                acc_sh.at[pl.ds(o, n)], out_hbm.at[q, pl.ds(o, n)]), sid)

    return k(g4n, src_st, dst_conv)


GB = 2000
NBLK = N // GB


def _tc_dinv(degp):
    def body(degp_ref, dinv_ref):
        deg = degp_ref[0, :, 0:1] + degp_ref[1, :, 0:1] + 1.0
        dinv_ref[...] = jnp.broadcast_to(jax.lax.rsqrt(deg), (N, QW))

    return pl.pallas_call(
        body, out_shape=jax.ShapeDtypeStruct((N, QW), jnp.float32))(degp)


def _split_quarters(g_ref, dv, h):
    for q in range(NQ):
        g_ref[q] = dv * h[:, q * QW:(q + 1) * QW]


def _tc_prep(x, W0, res0_W, res0_b, dinv):
    def body(x_ref, w_ref, rw_ref, rb_ref, dinv_ref, g_ref, res_ref):
        xv = x_ref[...]
        h = lax.dot_general(xv, w_ref[...], (((1,), (0,)), ((), ())),
                            precision=_HIGH, preferred_element_type=jnp.float32)
        _split_quarters(g_ref, dinv_ref[...], h)
        res_ref[...] = lax.dot_general(
            xv, rw_ref[...], (((1,), (1,)), ((), ())),
            precision=_HIGH, preferred_element_type=jnp.float32) + rb_ref[...]

    return pl.pallas_call(
        body,
        grid=(NBLK,),
        in_specs=[
            pl.BlockSpec((GB, IN_DIM), lambda i: (i, 0)),
            pl.BlockSpec((IN_DIM, HIDDEN), lambda i: (0, 0)),
            pl.BlockSpec((HIDDEN, IN_DIM), lambda i: (0, 0)),
            pl.BlockSpec((HIDDEN,), lambda i: (0,)),
            pl.BlockSpec((GB, QW), lambda i: (i, 0)),
        ],
        out_specs=[
            pl.BlockSpec((NQ, GB, QW), lambda i: (0, i, 0)),
            pl.BlockSpec((GB, HIDDEN), lambda i: (i, 0)),
        ],
        out_shape=[
            jax.ShapeDtypeStruct((NQ, N, QW), jnp.float32),
            jax.ShapeDtypeStruct((N, HIDDEN), jnp.float32),
        ])(x, W0, res0_W, res0_b, dinv)


def _tc_bn_stats(s, dinv, b):
    def body(s_ref, dinv_ref, b_ref, out0_ref, st_ref):
        dv = dinv_ref[...]
        out = jnp.concatenate([dv * s_ref[q] for q in range(NQ)],
                              axis=1) + b_ref[...]
        out0_ref[...] = out

        @pl.when(pl.program_id(0) == 0)
        def _():
            st_ref[...] = jnp.zeros_like(st_ref)

        st_ref[0:1, :] += jnp.sum(out, axis=0, keepdims=True)
        st_ref[1:2, :] += jnp.sum(out * out, axis=0, keepdims=True)

    return pl.pallas_call(
        body,
        grid=(NBLK,),
        in_specs=[
            pl.BlockSpec((NQ, GB, QW), lambda i: (0, i, 0)),
            pl.BlockSpec((GB, QW), lambda i: (i, 0)),
            pl.BlockSpec((HIDDEN,), lambda i: (0,)),
        ],
        out_specs=[
            pl.BlockSpec((GB, HIDDEN), lambda i: (i, 0)),
            pl.BlockSpec((2, HIDDEN), lambda i: (0, 0)),
        ],
        out_shape=[
            jax.ShapeDtypeStruct((N, HIDDEN), jnp.float32),
            jax.ShapeDtypeStruct((2, HIDDEN), jnp.float32),
        ])(s, dinv, b)


def _tc_bn_apply(out0, stats, bn_w, bn_b, res):
    def body(o_ref, st_ref, w_ref, bb_ref, res_ref, x1_ref):
        m = st_ref[0:1, :] * (1.0 / N)
        var = st_ref[1:2, :] * (1.0 / N) - m * m
        xb = (o_ref[...] - m) * jax.lax.rsqrt(var + EPS) * w_ref[...] + bb_ref[...]
        x1_ref[...] = jnp.maximum(xb, 0.0) + res_ref[...]

    return pl.pallas_call(
        body,
        grid=(NBLK,),
        in_specs=[
            pl.BlockSpec((GB, HIDDEN), lambda i: (i, 0)),
            pl.BlockSpec((2, HIDDEN), lambda i: (0, 0)),
            pl.BlockSpec((HIDDEN,), lambda i: (0,)),
            pl.BlockSpec((HIDDEN,), lambda i: (0,)),
            pl.BlockSpec((GB, HIDDEN), lambda i: (i, 0)),
        ],
        out_specs=pl.BlockSpec((GB, HIDDEN), lambda i: (i, 0)),
        out_shape=jax.ShapeDtypeStruct((N, HIDDEN), jnp.float32))(
            out0, stats, bn_w, bn_b, res)


def _tc_mm(x1, W1, dinv):
    def body(x_ref, w_ref, dinv_ref, g_ref):
        h = lax.dot_general(x_ref[...], w_ref[...], (((1,), (0,)), ((), ())),
                            precision=_HIGH, preferred_element_type=jnp.float32)
        _split_quarters(g_ref, dinv_ref[...], h)

    return pl.pallas_call(
        body,
        grid=(NBLK,),
        in_specs=[
            pl.BlockSpec((GB, HIDDEN), lambda i: (i, 0)),
            pl.BlockSpec((HIDDEN, HIDDEN), lambda i: (0, 0)),
            pl.BlockSpec((GB, QW), lambda i: (i, 0)),
        ],
        out_specs=pl.BlockSpec((NQ, GB, QW), lambda i: (0, i, 0)),
        out_shape=jax.ShapeDtypeStruct((NQ, N, QW), jnp.float32))(x1, W1, dinv)


def _tc_head(x2, Wh, bh, batch2d):
    def body(x2_ref, wh_ref, bh_ref, b2_ref, out_ref):
        logits = lax.dot_general(
            x2_ref[...], wh_ref[...], (((1,), (1,)), ((), ())),
            precision=_HIGH, preferred_element_type=jnp.float32) + bh_ref[...]
        seg = lax.broadcasted_iota(jnp.int32, (NUM_GRAPHS, N), 0)
        P = (b2_ref[...] == seg).astype(jnp.float32)
        pooled = lax.dot_general(P, logits, (((1,), (0,)), ((), ())),
                                 precision=_HIGH,
                                 preferred_element_type=jnp.float32)
        cnt = jnp.clip(jnp.sum(P, axis=1, keepdims=True), 1.0, None)
        out_ref[...] = pooled / cnt

    return pl.pallas_call(
        body,
        out_shape=jax.ShapeDtypeStruct((NUM_GRAPHS, NUM_CLASSES), jnp.float32))(
            x2, Wh, bh, batch2d)


def kernel(x, edge_index, batch, W0, b0, bn0_w, bn0_b, res0_W, res0_b,
           W1, b1, bn1_w, bn1_b, Wh, bh):
    src = edge_index[0].astype(jnp.int32)
    dst = edge_index[1].astype(jnp.int32)
    src_st = jnp.stack([src + q * N for q in range(NQ)]).reshape(
        NQ, NS, NCHUNK // NS, CH)
    dst_conv = dst.reshape(NS, NCHUNK // NS, CH)
    dst_deg = dst.reshape(NC * NS, NCHUNK // (NC * NS), CH)
    zeros8 = jnp.zeros((N, 8), jnp.float32)
    ones_ch = jnp.ones((CH, 8), jnp.float32)

    degp = _sc_degree(dst_deg, zeros8, ones_ch)
    dinv = _tc_dinv(degp)
    g0, res = _tc_prep(x, W0, res0_W, res0_b, dinv)
    s0 = _sc_conv(g0.reshape(NQ * N, QW), src_st, dst_conv)
    out0, st0 = _tc_bn_stats(s0, dinv, b0)
    x1 = _tc_bn_apply(out0, st0, bn0_w, bn0_b, res)
    g1 = _tc_mm(x1, W1, dinv)
    s1 = _sc_conv(g1.reshape(NQ * N, QW), src_st, dst_conv)
    out1, st1 = _tc_bn_stats(s1, dinv, b1)
    x2 = _tc_bn_apply(out1, st1, bn1_w, bn1_b, x1)
    return _tc_head(x2, Wh, bh, batch.astype(jnp.int32).reshape(1, N))
